# Initial kernel scaffold; baseline (speedup 1.0000x reference)
import jax
import jax.numpy as jnp
from jax.experimental import pallas as pl


def kernel(feature, text_feature, centers, text_centers, position, pm_camid, pm_vid, camid):
    raise NotImplementedError("write your pallas kernel here")


# speedup vs baseline: 37.3358x; 37.3358x over previous
import jax
import jax.numpy as jnp
from jax import lax
from jax.experimental import pallas as pl

_SCALE = 10.0
_K = 10
_TEMP = 0.5


def _align_body(feat_ref, txt_ref, aligned_ref, kl_ref):
    a = feat_ref[0]
    t = txt_ref[0]
    b = a.shape[0]
    an = jnp.sqrt(jnp.sum(a * a, axis=1, keepdims=True))
    a = a / jnp.maximum(an, 1e-12)
    tn = jnp.sqrt(jnp.sum(t * t, axis=1, keepdims=True))
    t = t / jnp.maximum(tn, 1e-12)
    si = lax.dot_general(a, a, (((1,), (1,)), ((), ())),
                         precision=lax.Precision.HIGHEST,
                         preferred_element_type=jnp.float32) * (1.0 / _TEMP)
    st = lax.dot_general(t, t, (((1,), (1,)), ((), ())),
                         precision=lax.Precision.HIGHEST,
                         preferred_element_type=jnp.float32) * (1.0 / _TEMP)

    def logsoftmax(s):
        m = jnp.max(s, axis=1, keepdims=True)
        e = jnp.exp(s - m)
        lse = jnp.log(jnp.sum(e, axis=1, keepdims=True)) + m
        return s - lse

    li = logsoftmax(si)
    lt = logsoftmax(st)
    pi = jnp.exp(li)
    pt = jnp.exp(lt)
    kl1 = jnp.sum(pt * (lt - li)) / b
    kl2 = jnp.sum(pi * (li - lt)) / b
    kl_ref[...] = jnp.broadcast_to(0.5 * (kl1 + kl2), (1, 1, 1))
    aligned_ref[0] = a + (t - a) * 0.1


def _knn_body(feat_ref, cen_ref, pos_ref, pmv0_ref, pmv1_ref,
              posvid_ref, rl_ref):
    a = feat_ref[0]
    c = cen_ref[0]
    r, ddim = a.shape
    n = c.shape[0]
    pos = pos_ref[0, 0, :][:, None]

    rn = jnp.sum(a * a, axis=1, keepdims=True)
    cn = lax.dot_general(jnp.ones((1, ddim), jnp.float32), c * c,
                         (((1,), (1,)), ((), ())),
                         precision=lax.Precision.HIGHEST,
                         preferred_element_type=jnp.float32)
    ac = lax.dot_general(a, c, (((1,), (1,)), ((), ())),
                         precision=lax.Precision.DEFAULT,
                         preferred_element_type=jnp.float32)
    d = rn + cn - 2.0 * ac

    colid = lax.broadcasted_iota(jnp.int32, (r, n), 1)
    d = jnp.where(colid == pos, jnp.inf, d)
    y = jnp.log(jnp.sum(jnp.exp(-_SCALE * d), axis=1))

    vid = jnp.where(colid < pos, pmv0_ref[0][None, :], pmv1_ref[0][None, :])
    packed = colid * 1024 + vid

    xsum = jnp.zeros((r, 1), jnp.float32)
    pv = jnp.zeros((r, 128), jnp.int32)
    lane = lax.broadcasted_iota(jnp.int32, (r, 128), 1)
    for k in range(_K):
        m = jnp.min(d, axis=1, keepdims=True)
        key = jnp.min(jnp.where(d == m, packed, jnp.int32(2 ** 30)),
                      axis=1, keepdims=True)
        idx = key // 1024
        v = key - idx * 1024
        xsum = xsum + jnp.exp(-_SCALE * m)
        pv = jnp.where(lane == k, v, pv)
        d = jnp.where(colid == idx, jnp.inf, d)
    x = jnp.log(xsum[:, 0])
    rl_ref[0, 0, :] = y - x
    posvid_ref[0] = pv


def kernel(feature, text_feature, centers, text_centers, position,
           pm_camid, pm_vid, camid):
    p, b, dd = feature.shape
    n = centers.shape[1]
    txt = jnp.transpose(text_feature, (1, 0, 2))

    aligned, klp = pl.pallas_call(
        _align_body,
        grid=(p,),
        in_specs=[
            pl.BlockSpec((1, b, dd), lambda i: (i, 0, 0)),
            pl.BlockSpec((1, b, dd), lambda i: (i, 0, 0)),
        ],
        out_specs=[
            pl.BlockSpec((1, b, dd), lambda i: (i, 0, 0)),
            pl.BlockSpec((1, 1, 1), lambda i: (i, 0, 0)),
        ],
        out_shape=[
            jax.ShapeDtypeStruct((p, b, dd), jnp.float32),
            jax.ShapeDtypeStruct((p, 1, 1), jnp.float32),
        ],
    )(feature, txt)

    r = 128 if b % 128 == 0 else b
    nb = b // r
    posr = position.reshape(nb, 1, r)
    pmv0 = pm_vid.reshape(1, n)
    pmv1 = jnp.concatenate([pm_vid[:1], pm_vid[:-1]]).reshape(1, n)

    posvid_pad, rowloss = pl.pallas_call(
        _knn_body,
        grid=(p, nb),
        in_specs=[
            pl.BlockSpec((1, r, dd), lambda i, j: (i, j, 0)),
            pl.BlockSpec((1, n, dd), lambda i, j: (i, 0, 0)),
            pl.BlockSpec((1, 1, r), lambda i, j: (j, 0, 0)),
            pl.BlockSpec((1, n), lambda i, j: (0, 0)),
            pl.BlockSpec((1, n), lambda i, j: (0, 0)),
        ],
        out_specs=[
            pl.BlockSpec((1, r, 128), lambda i, j: (i, j, 0)),
            pl.BlockSpec((1, 1, r), lambda i, j: (i * nb + j, 0, 0)),
        ],
        out_shape=[
            jax.ShapeDtypeStruct((p, b, 128), jnp.int32),
            jax.ShapeDtypeStruct((p * nb, 1, r), jnp.float32),
        ],
    )(aligned, centers, posr, pmv0, pmv1)

    lp = jnp.sum(rowloss.reshape(p, b), axis=1) / b
    lp = jnp.where(jnp.isnan(lp), 0.0, lp)
    loss = jnp.sum(lp) / p + 0.5 * jnp.sum(klp)
    return loss, posvid_pad[:, :, :_K]


# speedup vs baseline: 42.2302x; 1.1311x over previous
import jax
import jax.numpy as jnp
from jax import lax
from jax.experimental import pallas as pl

_SCALE = 10.0
_K = 10
_TEMP = 0.5


def _align_body(feat_ref, txt_ref, aligned_ref, kl_ref):
    a = feat_ref[0]
    t = txt_ref[0]
    b = a.shape[0]
    an = jnp.sqrt(jnp.sum(a * a, axis=1, keepdims=True))
    a = a / jnp.maximum(an, 1e-12)
    tn = jnp.sqrt(jnp.sum(t * t, axis=1, keepdims=True))
    t = t / jnp.maximum(tn, 1e-12)
    si = lax.dot_general(a, a, (((1,), (1,)), ((), ())),
                         precision=lax.Precision.HIGHEST,
                         preferred_element_type=jnp.float32) * (1.0 / _TEMP)
    st = lax.dot_general(t, t, (((1,), (1,)), ((), ())),
                         precision=lax.Precision.HIGHEST,
                         preferred_element_type=jnp.float32) * (1.0 / _TEMP)

    def logsoftmax(s):
        m = jnp.max(s, axis=1, keepdims=True)
        e = jnp.exp(s - m)
        lse = jnp.log(jnp.sum(e, axis=1, keepdims=True)) + m
        return s - lse

    li = logsoftmax(si)
    lt = logsoftmax(st)
    pi = jnp.exp(li)
    pt = jnp.exp(lt)
    kl1 = jnp.sum(pt * (lt - li)) / b
    kl2 = jnp.sum(pi * (li - lt)) / b
    kl_ref[...] = jnp.broadcast_to(0.5 * (kl1 + kl2), (1, 1, 1))
    aligned_ref[0] = a + (t - a) * 0.1


def _knn_body(feat_ref, cen_ref, pos_ref, pmv0_ref, pmv1_ref,
              posvid_ref, rl_ref):
    a = feat_ref[0]
    c = cen_ref[0]
    r, ddim = a.shape
    n = c.shape[0]
    h = n // 2
    posf = pos_ref[0, 0, :][:, None]

    rn = jnp.sum(a * a, axis=1, keepdims=True)
    cn = lax.dot_general(jnp.ones((1, ddim), jnp.float32), c * c,
                         (((1,), (1,)), ((), ())),
                         precision=lax.Precision.HIGHEST,
                         preferred_element_type=jnp.float32)
    ac = lax.dot_general(a, c, (((1,), (1,)), ((), ())),
                         precision=lax.Precision.DEFAULT,
                         preferred_element_type=jnp.float32)
    colf = lax.broadcasted_iota(jnp.int32, (r, n), 1).astype(jnp.float32)
    d = jnp.where(colf == posf, jnp.inf, rn + cn - 2.0 * ac)
    y = jnp.log(jnp.sum(jnp.exp(-_SCALE * d), axis=1))

    vid = jnp.where(colf < posf, pmv0_ref[0][None, :], pmv1_ref[0][None, :])
    pk = colf * 1024.0 + vid

    xsum = jnp.zeros((r, 1), jnp.float32)
    pv = jnp.zeros((r, 128), jnp.int32)
    lane = lax.broadcasted_iota(jnp.int32, (r, 128), 1)
    for k in range(_K):
        m = jnp.min(d, axis=1, keepdims=True)
        key = jnp.min(jnp.where(d == m, pk, jnp.inf),
                      axis=1, keepdims=True)
        xsum = xsum + jnp.exp(-_SCALE * m)
        idxf = jnp.floor(key * (1.0 / 1024.0))
        v = key - idxf * 1024.0
        pv = jnp.where(lane == k, v.astype(jnp.int32), pv)
        d = jnp.where(colf == idxf, jnp.inf, d)
    x = jnp.log(xsum[:, 0])
    rl_ref[0, 0, :] = y - x
    posvid_ref[0] = pv


def kernel(feature, text_feature, centers, text_centers, position,
           pm_camid, pm_vid, camid):
    p, b, dd = feature.shape
    n = centers.shape[1]
    txt = jnp.transpose(text_feature, (1, 0, 2))

    aligned, klp = pl.pallas_call(
        _align_body,
        grid=(p,),
        in_specs=[
            pl.BlockSpec((1, b, dd), lambda i: (i, 0, 0)),
            pl.BlockSpec((1, b, dd), lambda i: (i, 0, 0)),
        ],
        out_specs=[
            pl.BlockSpec((1, b, dd), lambda i: (i, 0, 0)),
            pl.BlockSpec((1, 1, 1), lambda i: (i, 0, 0)),
        ],
        out_shape=[
            jax.ShapeDtypeStruct((p, b, dd), jnp.float32),
            jax.ShapeDtypeStruct((p, 1, 1), jnp.float32),
        ],
    )(feature, txt)

    r = 128 if b % 128 == 0 else b
    nb = b // r
    posr = position.astype(jnp.float32).reshape(nb, 1, r)
    pmv0 = pm_vid.astype(jnp.float32).reshape(1, n)
    pmv1 = jnp.concatenate([pm_vid[:1], pm_vid[:-1]]).astype(
        jnp.float32).reshape(1, n)

    posvid_pad, rowloss = pl.pallas_call(
        _knn_body,
        grid=(p, nb),
        in_specs=[
            pl.BlockSpec((1, r, dd), lambda i, j: (i, j, 0)),
            pl.BlockSpec((1, n, dd), lambda i, j: (i, 0, 0)),
            pl.BlockSpec((1, 1, r), lambda i, j: (j, 0, 0)),
            pl.BlockSpec((1, n), lambda i, j: (0, 0)),
            pl.BlockSpec((1, n), lambda i, j: (0, 0)),
        ],
        out_specs=[
            pl.BlockSpec((1, r, 128), lambda i, j: (i, j, 0)),
            pl.BlockSpec((1, 1, r), lambda i, j: (i * nb + j, 0, 0)),
        ],
        out_shape=[
            jax.ShapeDtypeStruct((p, b, 128), jnp.int32),
            jax.ShapeDtypeStruct((p * nb, 1, r), jnp.float32),
        ],
    )(aligned, centers, posr, pmv0, pmv1)

    lp = jnp.sum(rowloss.reshape(p, b), axis=1) / b
    lp = jnp.where(jnp.isnan(lp), 0.0, lp)
    loss = jnp.sum(lp) / p + 0.5 * jnp.sum(klp)
    return loss, posvid_pad[:, :, :_K]


# speedup vs baseline: 43.5401x; 1.0310x over previous
import jax
import jax.numpy as jnp
from jax import lax
from jax.experimental import pallas as pl

_SCALE = 10.0
_K = 10
_TEMP = 0.5


def _align_body(feat_ref, txt_ref, aligned_ref, kl_ref):
    a = feat_ref[0]
    t = txt_ref[0]
    b = a.shape[0]
    an = jnp.sqrt(jnp.sum(a * a, axis=1, keepdims=True))
    a = a / jnp.maximum(an, 1e-12)
    tn = jnp.sqrt(jnp.sum(t * t, axis=1, keepdims=True))
    t = t / jnp.maximum(tn, 1e-12)
    si = lax.dot_general(a, a, (((1,), (1,)), ((), ())),
                         precision=lax.Precision.HIGHEST,
                         preferred_element_type=jnp.float32) * (1.0 / _TEMP)
    st = lax.dot_general(t, t, (((1,), (1,)), ((), ())),
                         precision=lax.Precision.HIGHEST,
                         preferred_element_type=jnp.float32) * (1.0 / _TEMP)

    def logsoftmax(s):
        m = jnp.max(s, axis=1, keepdims=True)
        e = jnp.exp(s - m)
        lse = jnp.log(jnp.sum(e, axis=1, keepdims=True)) + m
        return s - lse

    li = logsoftmax(si)
    lt = logsoftmax(st)
    pi = jnp.exp(li)
    pt = jnp.exp(lt)
    kl1 = jnp.sum(pt * (lt - li)) / b
    kl2 = jnp.sum(pi * (li - lt)) / b
    kl_ref[...] = jnp.broadcast_to(0.5 * (kl1 + kl2), (1, 1, 1))
    aligned_ref[0] = a + (t - a) * 0.1


def _knn_body(feat_ref, cen_ref, pos_ref, pmv0_ref, pmv1_ref,
              posvid_ref, rl_ref):
    a = feat_ref[0]
    c = cen_ref[0]
    r, ddim = a.shape
    n = c.shape[0]
    h = n // 2
    posf = pos_ref[0, 0, :][:, None]

    rn = jnp.sum(a * a, axis=1, keepdims=True)
    cn = lax.dot_general(jnp.ones((1, ddim), jnp.float32), c * c,
                         (((1,), (1,)), ((), ())),
                         precision=lax.Precision.HIGHEST,
                         preferred_element_type=jnp.float32)
    ac = lax.dot_general(a, c, (((1,), (1,)), ((), ())),
                         precision=lax.Precision.DEFAULT,
                         preferred_element_type=jnp.float32)
    colf = lax.broadcasted_iota(jnp.int32, (r, n), 1).astype(jnp.float32)
    d = jnp.where(colf == posf, jnp.inf, rn + cn - 2.0 * ac)
    y = jnp.log(jnp.sum(jnp.exp(-_SCALE * d), axis=1))

    vid = jnp.where(colf < posf, pmv0_ref[0][None, :], pmv1_ref[0][None, :])
    pk = colf * 1024.0 + vid

    d_lo, d_hi = d[:, :h], d[:, h:]
    p_lo, p_hi = pk[:, :h], pk[:, h:]
    swap = d_hi < d_lo
    f1 = jnp.where(swap, d_hi, d_lo)
    f2 = jnp.where(swap, d_lo, d_hi)
    p1 = jnp.where(swap, p_hi, p_lo)
    p2 = jnp.where(swap, p_lo, p_hi)

    xsum = jnp.zeros((r, 1), jnp.float32)
    pv = jnp.zeros((r, 128), jnp.int32)
    lane = lax.broadcasted_iota(jnp.int32, (r, 128), 1)
    m = jnp.min(f1, axis=1, keepdims=True)
    for k in range(_K):
        key = jnp.min(jnp.where(f1 == m, p1, jnp.inf),
                      axis=1, keepdims=True)
        xsum = xsum + jnp.exp(-_SCALE * m)
        v = key - jnp.floor(key * (1.0 / 1024.0)) * 1024.0
        pv = jnp.where(lane == k, v.astype(jnp.int32), pv)
        cond = p1 == key
        f1 = jnp.where(cond, f2, f1)
        p1 = jnp.where(cond, p2, p1)
        f2 = jnp.where(cond, jnp.inf, f2)
        if k < _K - 1:
            m = jnp.min(f1, axis=1, keepdims=True)
    x = jnp.log(xsum[:, 0])
    rl_ref[0, 0, :] = y - x
    posvid_ref[0] = pv


def kernel(feature, text_feature, centers, text_centers, position,
           pm_camid, pm_vid, camid):
    p, b, dd = feature.shape
    n = centers.shape[1]
    txt = jnp.transpose(text_feature, (1, 0, 2))

    aligned, klp = pl.pallas_call(
        _align_body,
        grid=(p,),
        in_specs=[
            pl.BlockSpec((1, b, dd), lambda i: (i, 0, 0)),
            pl.BlockSpec((1, b, dd), lambda i: (i, 0, 0)),
        ],
        out_specs=[
            pl.BlockSpec((1, b, dd), lambda i: (i, 0, 0)),
            pl.BlockSpec((1, 1, 1), lambda i: (i, 0, 0)),
        ],
        out_shape=[
            jax.ShapeDtypeStruct((p, b, dd), jnp.float32),
            jax.ShapeDtypeStruct((p, 1, 1), jnp.float32),
        ],
    )(feature, txt)

    r = 128 if b % 128 == 0 else b
    nb = b // r
    posr = position.astype(jnp.float32).reshape(nb, 1, r)
    pmv0 = pm_vid.astype(jnp.float32).reshape(1, n)
    pmv1 = jnp.concatenate([pm_vid[:1], pm_vid[:-1]]).astype(
        jnp.float32).reshape(1, n)

    posvid_pad, rowloss = pl.pallas_call(
        _knn_body,
        grid=(p, nb),
        in_specs=[
            pl.BlockSpec((1, r, dd), lambda i, j: (i, j, 0)),
            pl.BlockSpec((1, n, dd), lambda i, j: (i, 0, 0)),
            pl.BlockSpec((1, 1, r), lambda i, j: (j, 0, 0)),
            pl.BlockSpec((1, n), lambda i, j: (0, 0)),
            pl.BlockSpec((1, n), lambda i, j: (0, 0)),
        ],
        out_specs=[
            pl.BlockSpec((1, r, 128), lambda i, j: (i, j, 0)),
            pl.BlockSpec((1, 1, r), lambda i, j: (i * nb + j, 0, 0)),
        ],
        out_shape=[
            jax.ShapeDtypeStruct((p, b, 128), jnp.int32),
            jax.ShapeDtypeStruct((p * nb, 1, r), jnp.float32),
        ],
    )(aligned, centers, posr, pmv0, pmv1)

    lp = jnp.sum(rowloss.reshape(p, b), axis=1) / b
    lp = jnp.where(jnp.isnan(lp), 0.0, lp)
    loss = jnp.sum(lp) / p + 0.5 * jnp.sum(klp)
    return loss, posvid_pad[:, :, :_K]


# speedup vs baseline: 47.3262x; 1.0870x over previous
import jax
import jax.numpy as jnp
from jax import lax
from jax.experimental import pallas as pl

_SCALE = 10.0
_K = 10
_TEMP = 0.5


def _align_body(feat_ref, txt_ref, aligned_ref, kl_ref):
    a = feat_ref[0]
    t = txt_ref[0]
    b = a.shape[0]
    an = jnp.sqrt(jnp.sum(a * a, axis=1, keepdims=True))
    a = a / jnp.maximum(an, 1e-12)
    tn = jnp.sqrt(jnp.sum(t * t, axis=1, keepdims=True))
    t = t / jnp.maximum(tn, 1e-12)
    si = lax.dot_general(a, a, (((1,), (1,)), ((), ())),
                         precision=lax.Precision.HIGHEST,
                         preferred_element_type=jnp.float32) * (1.0 / _TEMP)
    st = lax.dot_general(t, t, (((1,), (1,)), ((), ())),
                         precision=lax.Precision.HIGHEST,
                         preferred_element_type=jnp.float32) * (1.0 / _TEMP)

    def logsoftmax(s):
        m = jnp.max(s, axis=1, keepdims=True)
        e = jnp.exp(s - m)
        lse = jnp.log(jnp.sum(e, axis=1, keepdims=True)) + m
        return s - lse

    li = logsoftmax(si)
    lt = logsoftmax(st)
    pi = jnp.exp(li)
    pt = jnp.exp(lt)
    kl1 = jnp.sum(pt * (lt - li)) / b
    kl2 = jnp.sum(pi * (li - lt)) / b
    kl_ref[...] = jnp.broadcast_to(0.5 * (kl1 + kl2), (1, 1, 1))
    aligned_ref[0] = a + (t - a) * 0.1


def _knn_body(feat_ref, cen_ref, pos_ref, pmv0_ref, pmv1_ref,
              posvid_ref, rl_ref):
    a = feat_ref[0]
    c = cen_ref[0]
    r, ddim = a.shape
    n = c.shape[0]
    h = n // 2
    posf = pos_ref[0, 0, :][:, None]

    rn = jnp.sum(a * a, axis=1, keepdims=True)
    cn = lax.dot_general(jnp.ones((1, ddim), jnp.float32), c * c,
                         (((1,), (1,)), ((), ())),
                         precision=lax.Precision.HIGHEST,
                         preferred_element_type=jnp.float32)
    ac = lax.dot_general(a, c, (((1,), (1,)), ((), ())),
                         precision=lax.Precision.DEFAULT,
                         preferred_element_type=jnp.float32)
    colf = lax.broadcasted_iota(jnp.int32, (r, n), 1).astype(jnp.float32)
    d = jnp.where(colf == posf, jnp.inf, rn + cn - 2.0 * ac)
    y = jnp.log(jnp.sum(jnp.exp(-_SCALE * d), axis=1))

    vid = jnp.where(colf < posf, pmv0_ref[0][None, :], pmv1_ref[0][None, :])
    pk = colf * 1024.0 + vid

    d_lo, d_hi = d[:, :h], d[:, h:]
    p_lo, p_hi = pk[:, :h], pk[:, h:]
    swap = d_hi < d_lo
    f1 = jnp.where(swap, d_hi, d_lo)
    f2 = jnp.where(swap, d_lo, d_hi)
    p1 = jnp.where(swap, p_hi, p_lo)
    p2 = jnp.where(swap, p_lo, p_hi)

    xsum = jnp.zeros((r, 1), jnp.float32)
    pv = jnp.zeros((r, 128), jnp.int32)
    lane = lax.broadcasted_iota(jnp.int32, (r, 128), 1)
    m = jnp.min(f1, axis=1, keepdims=True)
    for k in range(_K):
        key = jnp.min(jnp.where(f1 == m, p1, jnp.inf),
                      axis=1, keepdims=True)
        xsum = xsum + jnp.exp(-_SCALE * m)
        v = key - jnp.floor(key * (1.0 / 1024.0)) * 1024.0
        pv = jnp.where(lane == k, v.astype(jnp.int32), pv)
        cond = p1 == key
        f1 = jnp.where(cond, f2, f1)
        p1 = jnp.where(cond, p2, p1)
        f2 = jnp.where(cond, jnp.inf, f2)
        if k < _K - 1:
            m = jnp.min(f1, axis=1, keepdims=True)
    x = jnp.log(xsum[:, 0])
    rl_ref[0, 0, :] = y - x
    posvid_ref[0] = pv


def kernel(feature, text_feature, centers, text_centers, position,
           pm_camid, pm_vid, camid):
    p, b, dd = feature.shape
    n = centers.shape[1]
    txt = jnp.transpose(text_feature, (1, 0, 2))

    aligned, klp = pl.pallas_call(
        _align_body,
        grid=(p,),
        in_specs=[
            pl.BlockSpec((1, b, dd), lambda i: (i, 0, 0)),
            pl.BlockSpec((1, b, dd), lambda i: (i, 0, 0)),
        ],
        out_specs=[
            pl.BlockSpec((1, b, dd), lambda i: (i, 0, 0)),
            pl.BlockSpec((1, 1, 1), lambda i: (i, 0, 0)),
        ],
        out_shape=[
            jax.ShapeDtypeStruct((p, b, dd), jnp.float32),
            jax.ShapeDtypeStruct((p, 1, 1), jnp.float32),
        ],
    )(feature, txt)

    r = 256 if b % 256 == 0 else b
    nb = b // r
    posr = position.astype(jnp.float32).reshape(nb, 1, r)
    pmv0 = pm_vid.astype(jnp.float32).reshape(1, n)
    pmv1 = jnp.concatenate([pm_vid[:1], pm_vid[:-1]]).astype(
        jnp.float32).reshape(1, n)

    posvid_pad, rowloss = pl.pallas_call(
        _knn_body,
        grid=(p, nb),
        in_specs=[
            pl.BlockSpec((1, r, dd), lambda i, j: (i, j, 0)),
            pl.BlockSpec((1, n, dd), lambda i, j: (i, 0, 0)),
            pl.BlockSpec((1, 1, r), lambda i, j: (j, 0, 0)),
            pl.BlockSpec((1, n), lambda i, j: (0, 0)),
            pl.BlockSpec((1, n), lambda i, j: (0, 0)),
        ],
        out_specs=[
            pl.BlockSpec((1, r, 128), lambda i, j: (i, j, 0)),
            pl.BlockSpec((1, 1, r), lambda i, j: (i * nb + j, 0, 0)),
        ],
        out_shape=[
            jax.ShapeDtypeStruct((p, b, 128), jnp.int32),
            jax.ShapeDtypeStruct((p * nb, 1, r), jnp.float32),
        ],
    )(aligned, centers, posr, pmv0, pmv1)

    lp = jnp.sum(rowloss.reshape(p, b), axis=1) / b
    lp = jnp.where(jnp.isnan(lp), 0.0, lp)
    loss = jnp.sum(lp) / p + 0.5 * jnp.sum(klp)
    return loss, posvid_pad[:, :, :_K]
